# baseline (speedup 1.0000x reference)
import jax
import jax.numpy as jnp
from jax import lax
from jax.experimental import pallas as pl
from jax.experimental.pallas import tpu as pltpu
from jax.experimental.pallas import tpu_sc as plsc

_BATCH = 16384
_D = 64
_W = 2 * _D
_LANES = 16

_info = plsc.get_sparse_core_info()
_NC, _NS = _info.num_cores, _info.num_subcores
_NW = _NC * _NS
_BPW = _BATCH // _NW
_CHUNK = 128
_NCHUNK = _BPW // _CHUNK
_GPC = _CHUNK // _LANES


def _body(user_hbm, item_hbm, uf_hbm, if_hbm, out_hbm,
          idx_ou, idx_oi, idx_du, idx_di, rows_u, rows_i, out_v, sem):
    wid = lax.axis_index("s") * _NC + lax.axis_index("c")
    base = wid * _BPW

    for c in range(_NCHUNK):
        pltpu.sync_copy(user_hbm.at[pl.ds(base + c * _CHUNK, _CHUNK)],
                        idx_ou.at[c])
        pltpu.sync_copy(item_hbm.at[pl.ds(base + c * _CHUNK, _CHUNK)],
                        idx_oi.at[c])

    for c in range(_NCHUNK):
        def shift(k, carry, c=c):
            s = pl.ds(k * _LANES, _LANES)
            idx_du[c, s] = lax.shift_right_logical(idx_ou[c, s], 1)
            idx_di[c, s] = lax.shift_right_logical(idx_oi[c, s], 1)
            return carry
        lax.fori_loop(0, _GPC, shift, 0)

    def fire(c, buf):
        return (
            pltpu.async_copy(uf_hbm.at[idx_du.at[c]], rows_u.at[buf], sem),
            pltpu.async_copy(if_hbm.at[idx_di.at[c]], rows_i.at[buf], sem),
        )

    iota = lax.iota(jnp.int32, _LANES)
    pending = fire(0, 0)
    for c in range(_NCHUNK):
        nxt = fire(c + 1, (c + 1) % 2) if c + 1 < _NCHUNK else None
        pending[0].wait()
        pending[1].wait()
        pending = nxt
        ru = rows_u.at[c % 2]
        ri = rows_i.at[c % 2]

        def group(k, carry, c=c, ru=ru, ri=ri):
            s = pl.ds(k * _LANES, _LANES)
            row = k * _LANES + iota
            col_u = (idx_ou[c, s] & 1) * _D
            col_i = (idx_oi[c, s] & 1) * _D
            acc = jnp.zeros((_LANES,), jnp.float32)
            for d in range(_D):
                u = plsc.load_gather(ru, [row, col_u + d])
                v = plsc.load_gather(ri, [row, col_i + d])
                acc = acc + u * v
            out_v[pl.ds(c * _CHUNK + k * _LANES, _LANES)] = acc
            return carry

        lax.fori_loop(0, _GPC, group, 0)

    pltpu.sync_copy(out_v, out_hbm.at[pl.ds(base, _BPW)])


@jax.jit
def _run(user, item, user_factors, item_factors):
    uf2 = user_factors.reshape(-1, _W)
    if2 = item_factors.reshape(-1, _W)
    mesh = plsc.VectorSubcoreMesh(core_axis_name="c", subcore_axis_name="s")
    fn = pl.kernel(
        _body,
        mesh=mesh,
        out_type=jax.ShapeDtypeStruct((_BATCH,), jnp.float32),
        scratch_types=[
            pltpu.VMEM((_NCHUNK, _CHUNK), jnp.int32),
            pltpu.VMEM((_NCHUNK, _CHUNK), jnp.int32),
            pltpu.VMEM((_NCHUNK, _CHUNK), jnp.int32),
            pltpu.VMEM((_NCHUNK, _CHUNK), jnp.int32),
            pltpu.VMEM((2, _CHUNK, _W), jnp.float32),
            pltpu.VMEM((2, _CHUNK, _W), jnp.float32),
            pltpu.VMEM((_BPW,), jnp.float32),
            pltpu.SemaphoreType.DMA,
        ],
        compiler_params=pltpu.CompilerParams(needs_layout_passes=False),
    )
    return fn(user, item, uf2, if2)


def kernel(user, item, user_factors, item_factors):
    return _run(user, item, user_factors, item_factors)
